# baseline (speedup 1.0000x reference)
import functools

import jax
import jax.numpy as jnp
from jax import lax
from jax.experimental import pallas as pl
from jax.experimental.pallas import tpu as pltpu
from jax.experimental.pallas import tpu_sc as plsc

VOCAB = 1000000
EMBED = 32
NUM_CLASSES = 100
BATCH = 4096
HIST = 200

NUM_CORES = 2
NUM_SUBCORES = 16
NUM_WORKERS = NUM_CORES * NUM_SUBCORES
B_PER_W = BATCH // NUM_WORKERS
C0 = 104
C1 = HIST - C0
NBUF = 8

_SCALE = 1.0 / HIST


def _pool_body(ids_hbm, table_hbm, out_hbm, idx_v, buf0, buf1, pooled_v, *sems):
    wid = lax.axis_index("s") * NUM_CORES + lax.axis_index("c")
    base = wid * B_PER_W
    pltpu.sync_copy(ids_hbm.at[pl.ds(base, B_PER_W)], idx_v)

    def fire(r, b):
        pltpu.async_copy(
            table_hbm.at[idx_v.at[r, pl.ds(0, C0)]], buf0.at[b], sems[b]
        )
        pltpu.async_copy(
            table_hbm.at[idx_v.at[r, pl.ds(C0, C1)]], buf1.at[b], sems[b]
        )

    def drain(b):
        pltpu.make_async_copy(
            table_hbm.at[idx_v.at[0, pl.ds(0, C0)]], buf0.at[b], sems[b]
        ).wait()
        pltpu.make_async_copy(
            table_hbm.at[idx_v.at[0, pl.ds(C0, C1)]], buf1.at[b], sems[b]
        ).wait()

    def accumulate(buf, n, a0, a1):
        for i in range(n):
            a0[i % 4] = a0[i % 4] + buf[i, 0:16]
            a1[i % 4] = a1[i % 4] + buf[i, 16:32]
        return a0, a1

    for b in range(NBUF):
        fire(b, b)

    def group_body(k, carry):
        g = k * NBUF
        for b in range(NBUF):
            r = g + b
            drain(b)
            z = jnp.zeros((16,), jnp.float32)
            a0 = [z, z, z, z]
            a1 = [z, z, z, z]
            a0, a1 = accumulate(buf0.at[b], C0, a0, a1)
            a0, a1 = accumulate(buf1.at[b], C1, a0, a1)
            pooled_v[r, 0:16] = ((a0[0] + a0[1]) + (a0[2] + a0[3])) * _SCALE
            pooled_v[r, 16:32] = ((a1[0] + a1[1]) + (a1[2] + a1[3])) * _SCALE

            @pl.when(r + NBUF < B_PER_W)
            def _():
                fire(r + NBUF, b)

        return carry

    lax.fori_loop(0, B_PER_W // NBUF, group_body, 0)
    pltpu.sync_copy(pooled_v, out_hbm.at[pl.ds(base, B_PER_W)])


def _make_pool_kernel():
    mesh = plsc.VectorSubcoreMesh(
        core_axis_name="c",
        subcore_axis_name="s",
        num_cores=NUM_CORES,
        num_subcores=NUM_SUBCORES,
    )
    return pl.kernel(
        _pool_body,
        out_type=jax.ShapeDtypeStruct((BATCH, EMBED), jnp.float32),
        mesh=mesh,
        scratch_types=[
            pltpu.VMEM((B_PER_W, HIST), jnp.int32),
            pltpu.VMEM((NBUF, C0, EMBED), jnp.float32),
            pltpu.VMEM((NBUF, C1, EMBED), jnp.float32),
            pltpu.VMEM((B_PER_W, EMBED), jnp.float32),
        ]
        + [pltpu.SemaphoreType.DMA] * NBUF,
        compiler_params=pltpu.CompilerParams(use_tc_tiling_on_sc=False),
    )


def _linear_body(pooled_ref, w_ref, b_ref, out_ref):
    out_ref[...] = (
        jnp.dot(pooled_ref[...], w_ref[...], preferred_element_type=jnp.float32)
        + b_ref[...]
    )


def kernel(input_ids, emb_table, fc_w, fc_b):
    ids = input_ids.astype(jnp.int32)
    pooled = _make_pool_kernel()(ids, emb_table)
    out = pl.pallas_call(
        _linear_body,
        out_shape=jax.ShapeDtypeStruct((BATCH, NUM_CLASSES), jnp.float32),
    )(pooled, fc_w.T, fc_b[None, :])
    return out


# speedup vs baseline: 1.0033x; 1.0033x over previous
import functools

import jax
import jax.numpy as jnp
from jax import lax
from jax.experimental import pallas as pl
from jax.experimental.pallas import tpu as pltpu
from jax.experimental.pallas import tpu_sc as plsc

VOCAB = 1000000
EMBED = 32
NUM_CLASSES = 100
BATCH = 4096
HIST = 200

NUM_CORES = 2
NUM_SUBCORES = 16
NUM_WORKERS = NUM_CORES * NUM_SUBCORES
B_PER_W = BATCH // NUM_WORKERS
C0 = 104
C1 = HIST - C0
NBUF = 8

_SCALE = 1.0 / HIST


def _pool_body(ids_hbm, table_hbm, out_hbm, idx_v, buf0, buf1, pooled_v, *sems):
    wid = lax.axis_index("s") * NUM_CORES + lax.axis_index("c")
    base = wid * B_PER_W
    pltpu.sync_copy(ids_hbm.at[pl.ds(base * HIST, B_PER_W * HIST)], idx_v)

    def fire(r, b):
        off = pl.multiple_of(r * HIST, 8)
        pltpu.async_copy(table_hbm.at[idx_v.at[pl.ds(off, C0)]], buf0.at[b], sems[b])
        pltpu.async_copy(
            table_hbm.at[idx_v.at[pl.ds(off + C0, C1)]], buf1.at[b], sems[b]
        )

    def drain(b):
        pltpu.make_async_copy(
            table_hbm.at[idx_v.at[pl.ds(0, C0)]], buf0.at[b], sems[b]
        ).wait()
        pltpu.make_async_copy(
            table_hbm.at[idx_v.at[pl.ds(0, C1)]], buf1.at[b], sems[b]
        ).wait()

    def accumulate(buf, n, a0, a1):
        for i in range(n):
            a0[i % 4] = a0[i % 4] + buf[i, 0:16]
            a1[i % 4] = a1[i % 4] + buf[i, 16:32]
        return a0, a1

    for b in range(NBUF):
        fire(b, b)

    def group_body(k, carry):
        g = k * NBUF
        for b in range(NBUF):
            r = g + b
            drain(b)
            z = jnp.zeros((16,), jnp.float32)
            a0 = [z, z, z, z]
            a1 = [z, z, z, z]
            a0, a1 = accumulate(buf0.at[b], C0, a0, a1)
            a0, a1 = accumulate(buf1.at[b], C1, a0, a1)
            pooled_v[r, 0:16] = ((a0[0] + a0[1]) + (a0[2] + a0[3])) * _SCALE
            pooled_v[r, 16:32] = ((a1[0] + a1[1]) + (a1[2] + a1[3])) * _SCALE

            @pl.when(r + NBUF < B_PER_W)
            def _():
                fire(r + NBUF, b)

        return carry

    lax.fori_loop(0, B_PER_W // NBUF, group_body, 0)
    pltpu.sync_copy(pooled_v, out_hbm.at[pl.ds(base, B_PER_W)])


def _make_pool_kernel():
    mesh = plsc.VectorSubcoreMesh(
        core_axis_name="c",
        subcore_axis_name="s",
        num_cores=NUM_CORES,
        num_subcores=NUM_SUBCORES,
    )
    return pl.kernel(
        _pool_body,
        out_type=jax.ShapeDtypeStruct((BATCH, EMBED), jnp.float32),
        mesh=mesh,
        scratch_types=[
            pltpu.VMEM((B_PER_W * HIST,), jnp.int32),
            pltpu.VMEM((NBUF, C0, EMBED), jnp.float32),
            pltpu.VMEM((NBUF, C1, EMBED), jnp.float32),
            pltpu.VMEM((B_PER_W, EMBED), jnp.float32),
        ]
        + [pltpu.SemaphoreType.DMA] * NBUF,
        compiler_params=pltpu.CompilerParams(use_tc_tiling_on_sc=False),
    )


def _linear_body(pooled_ref, w_ref, b_ref, out_ref):
    out_ref[...] = (
        jnp.dot(pooled_ref[...], w_ref[...], preferred_element_type=jnp.float32)
        + b_ref[...]
    )


def kernel(input_ids, emb_table, fc_w, fc_b):
    ids_flat = jnp.minimum(input_ids.astype(jnp.int32), VOCAB - 1).reshape(-1)
    pooled = _make_pool_kernel()(ids_flat, emb_table)
    out = pl.pallas_call(
        _linear_body,
        out_shape=jax.ShapeDtypeStruct((BATCH, NUM_CLASSES), jnp.float32),
    )(pooled, fc_w.T, fc_b[None, :])
    return out


# speedup vs baseline: 1.0931x; 1.0895x over previous
import functools

import jax
import jax.numpy as jnp
from jax import lax
from jax.experimental import pallas as pl
from jax.experimental.pallas import tpu as pltpu
from jax.experimental.pallas import tpu_sc as plsc

VOCAB = 1000000
EMBED = 32
NUM_CLASSES = 100
BATCH = 4096
HIST = 200

NUM_CORES = 2
NUM_SUBCORES = 16
NUM_WORKERS = NUM_CORES * NUM_SUBCORES
B_PER_W = BATCH // NUM_WORKERS
LSEG = 10
NSEG = HIST // LSEG
JB = B_PER_W // 16

_SCALE = 1.0 / HIST


def _pool_body(idsT_hbm, table_hbm, out_hbm, idx_v, buf, pooled_v, sem0, sem1):
    wid = lax.axis_index("s") * NUM_CORES + lax.axis_index("c")
    base = wid * B_PER_W
    pltpu.sync_copy(idsT_hbm.at[:, pl.ds(base, B_PER_W)], idx_v)

    sems = (sem0, sem1)

    def zero_body(r, carry):
        z = jnp.zeros((16,), jnp.float32)
        pooled_v[r, 0:16] = z
        pooled_v[r, 16:32] = z
        return carry

    lax.fori_loop(0, B_PER_W, zero_body, 0)

    def fire(seg, p):
        for ll in range(LSEG):
            pltpu.async_copy(
                table_hbm.at[idx_v.at[seg * LSEG + ll]], buf.at[p, ll], sems[p]
            )

    def drain(p):
        for ll in range(LSEG):
            pltpu.make_async_copy(
                table_hbm.at[idx_v.at[0]], buf.at[p, ll], sems[p]
            ).wait()

    def process(p):
        def jb_body(jb, carry):
            jbase = jb * 16
            acc = []
            for r in range(16):
                acc.append(pooled_v[jbase + r, 0:16])
                acc.append(pooled_v[jbase + r, 16:32])
            for ll in range(LSEG):
                for r in range(16):
                    acc[2 * r] = acc[2 * r] + buf[p, ll, jbase + r, 0:16]
                    acc[2 * r + 1] = acc[2 * r + 1] + buf[p, ll, jbase + r, 16:32]
            for r in range(16):
                pooled_v[jbase + r, 0:16] = acc[2 * r]
                pooled_v[jbase + r, 16:32] = acc[2 * r + 1]
            return carry

        lax.fori_loop(0, JB, jb_body, 0)

    fire(0, 0)

    def pair_body(k, carry):
        s0 = 2 * k
        fire(s0 + 1, 1)
        drain(0)
        process(0)

        @pl.when(s0 + 2 < NSEG)
        def _():
            fire(s0 + 2, 0)

        drain(1)
        process(1)
        return carry

    lax.fori_loop(0, NSEG // 2, pair_body, 0)
    pltpu.sync_copy(pooled_v, out_hbm.at[pl.ds(base, B_PER_W)])


def _make_pool_kernel():
    mesh = plsc.VectorSubcoreMesh(
        core_axis_name="c",
        subcore_axis_name="s",
        num_cores=NUM_CORES,
        num_subcores=NUM_SUBCORES,
    )
    return pl.kernel(
        _pool_body,
        out_type=jax.ShapeDtypeStruct((BATCH, EMBED), jnp.float32),
        mesh=mesh,
        scratch_types=[
            pltpu.VMEM((HIST, B_PER_W), jnp.int32),
            pltpu.VMEM((2, LSEG, B_PER_W, EMBED), jnp.float32),
            pltpu.VMEM((B_PER_W, EMBED), jnp.float32),
            pltpu.SemaphoreType.DMA,
            pltpu.SemaphoreType.DMA,
        ],
        compiler_params=pltpu.CompilerParams(use_tc_tiling_on_sc=False),
    )


def _linear_body(pooled_ref, w_ref, b_ref, out_ref):
    out_ref[...] = (
        jnp.dot(pooled_ref[...], w_ref[...], preferred_element_type=jnp.float32)
        * _SCALE
        + b_ref[...]
    )


def kernel(input_ids, emb_table, fc_w, fc_b):
    ids_t = jnp.transpose(input_ids.astype(jnp.int32))
    pooled = _make_pool_kernel()(ids_t, emb_table)
    out = pl.pallas_call(
        _linear_body,
        out_shape=jax.ShapeDtypeStruct((BATCH, NUM_CLASSES), jnp.float32),
    )(pooled, fc_w.T, fc_b[None, :])
    return out
